# baseline (speedup 1.0000x reference)
import functools

import numpy as np
import jax
import jax.numpy as jnp
from jax import lax
from jax.experimental import pallas as pl
from jax.experimental.pallas import tpu as pltpu
from jax.experimental.pallas import tpu_sc as plsc

D = 128
NV = 50000
NB = 100000
NC, NS = 2, 16
NW = NC * NS
B_PAD = 102400
PER_W = B_PAD // NW
CHUNK = 128
N_CHUNKS = PER_W // CHUNK
NBUF = 3
LA = 2

_INV_PU = np.array([
    36, 58, 29, 7, 81, 105, 42, 113, 57, 115, 18, 3, 125, 93, 78, 102, 22,
    27, 10, 76, 60, 24, 95, 31, 87, 96, 127, 116, 92, 111, 101, 47, 40, 32,
    69, 28, 61, 122, 85, 37, 118, 51, 44, 34, 21, 79, 80, 73, 26, 119, 56,
    110, 52, 54, 1, 124, 67, 11, 41, 63, 12, 15, 23, 114, 121, 112, 45, 50,
    74, 108, 9, 75, 20, 48, 82, 86, 35, 38, 65, 59, 49, 55, 103, 97, 71, 33,
    5, 46, 4, 83, 106, 72, 70, 8, 0, 2, 98, 100, 84, 99, 25, 64, 94, 53,
    123, 13, 107, 43, 90, 6, 66, 89, 88, 17, 39, 77, 68, 104, 91, 126, 117,
    109, 14, 120, 19, 62, 16, 30], dtype=np.int32)
_INV_PI = np.array([
    102, 18, 20, 39, 35, 104, 13, 38, 87, 98, 82, 125, 103, 59, 33, 100,
    123, 26, 70, 42, 69, 99, 68, 90, 46, 56, 111, 63, 15, 121, 14, 126, 28,
    16, 110, 4, 113, 22, 65, 106, 57, 72, 54, 41, 62, 24, 48, 52, 29, 91,
    74, 107, 58, 21, 76, 124, 31, 12, 19, 0, 67, 79, 95, 17, 50, 45, 10, 96,
    120, 34, 23, 47, 73, 44, 92, 115, 32, 2, 75, 81, 117, 66, 97, 101, 105,
    53, 127, 83, 118, 108, 114, 71, 89, 36, 86, 1, 27, 11, 88, 77, 112, 85,
    5, 84, 49, 43, 60, 9, 37, 64, 8, 3, 109, 122, 55, 119, 61, 51, 30, 7,
    40, 80, 78, 116, 94, 25, 6, 93], dtype=np.int32)
_PERM23 = np.concatenate([_INV_PU + D, _INV_PI + 2 * D])


def _proj_body(u_ref, i_ref, w2_ref, w3_ref, up_ref, ip_ref):
    up_ref[...] = jnp.dot(u_ref[...], w2_ref[...],
                          preferred_element_type=jnp.float32)
    ip_ref[...] = jnp.dot(i_ref[...], w3_ref[...],
                          preferred_element_type=jnp.float32)


def _project_tables(user_feats, item_feats, w2p, w3p):
    blk = 10000
    return pl.pallas_call(
        _proj_body,
        grid=(NV // blk,),
        in_specs=[
            pl.BlockSpec((blk, D), lambda i: (i, 0)),
            pl.BlockSpec((blk, D), lambda i: (i, 0)),
            pl.BlockSpec((D, D), lambda i: (0, 0)),
            pl.BlockSpec((D, D), lambda i: (0, 0)),
        ],
        out_specs=[pl.BlockSpec((blk, D), lambda i: (i, 0)),
                   pl.BlockSpec((blk, D), lambda i: (i, 0))],
        out_shape=[jax.ShapeDtypeStruct((NV, D), jnp.float32),
                   jax.ShapeDtypeStruct((NV, D), jnp.float32)],
    )(user_feats, item_feats, w2p, w3p)


def _sc_gather_add(up, ipj, idx2):
    mesh = plsc.VectorSubcoreMesh(core_axis_name="c", subcore_axis_name="s")

    @functools.partial(
        pl.kernel,
        out_type=jax.ShapeDtypeStruct((B_PAD, D), jnp.float32),
        mesh=mesh,
        scratch_types=(
            [pltpu.VMEM((N_CHUNKS, CHUNK), jnp.int32),
             pltpu.VMEM((N_CHUNKS, CHUNK), jnp.int32),
             pltpu.VMEM((NBUF, CHUNK, D), jnp.float32),
             pltpu.VMEM((NBUF, CHUNK, D), jnp.float32)]
            + [pltpu.SemaphoreType.DMA] * (3 * NBUF)
        ),
    )
    def k(up_hbm, ip_hbm, idx_hbm, g_hbm, idxu, idxi, rows_u, rows_i,
          *sems):
        sem_u = sems[0:NBUF]
        sem_i = sems[NBUF:2 * NBUF]
        sem_o = sems[2 * NBUF:3 * NBUF]
        wid = lax.axis_index("s") * NC + lax.axis_index("c")
        base = wid * PER_W
        pltpu.sync_copy(idx_hbm.at[0, wid], idxu)
        pltpu.sync_copy(idx_hbm.at[1, wid], idxi)

        g = [None] * N_CHUNKS
        w = [None] * N_CHUNKS

        def issue_g(j):
            b = j % NBUF
            g[j] = (pltpu.async_copy(up_hbm.at[idxu.at[j]], rows_u.at[b],
                                     sem_u[b]),
                    pltpu.async_copy(ip_hbm.at[idxi.at[j]], rows_i.at[b],
                                     sem_i[b]))

        for j in range(LA):
            issue_g(j)
        for j in range(N_CHUNKS):
            b = j % NBUF
            g[j][0].wait()
            g[j][1].wait()
            au = rows_u.at[b]
            ai = rows_i.at[b]

            def add_row(r, carry):
                for kk in range(D // 16):
                    s = pl.ds(kk * 16, 16)
                    au[r, s] = au[r, s] + ai[r, s]
                return carry

            lax.fori_loop(0, CHUNK, add_row, 0)
            w[j] = pltpu.async_copy(
                au, g_hbm.at[pl.ds(base + j * CHUNK, CHUNK)], sem_o[b])
            if j + LA < N_CHUNKS:
                if j + LA - NBUF >= 0:
                    w[j + LA - NBUF].wait()
                issue_g(j + LA)
        for j in range(N_CHUNKS + LA - NBUF, N_CHUNKS):
            w[j].wait()

    return k(up, ipj, idx2)


def _final_body(r_ref, g_ref, w1_ref, o_ref):
    acc = jnp.dot(r_ref[...], w1_ref[...], preferred_element_type=jnp.float32)
    o_ref[...] = jnp.maximum(acc + g_ref[...], 0.0)


def _final(review, g, w1):
    blk = 5000
    return pl.pallas_call(
        _final_body,
        grid=(NB // blk,),
        in_specs=[
            pl.BlockSpec((blk, D), lambda i: (i, 0)),
            pl.BlockSpec((blk, D), lambda i: (i, 0)),
            pl.BlockSpec((D, D), lambda i: (0, 0)),
        ],
        out_specs=pl.BlockSpec((blk, D), lambda i: (i, 0)),
        out_shape=jax.ShapeDtypeStruct((NB, D), jnp.float32),
    )(review, g, w1)


def _pad_idx2(user_idx, item_idx):
    both = jnp.stack([user_idx.astype(jnp.int32), item_idx.astype(jnp.int32)])
    rows = jnp.arange(B_PAD, dtype=jnp.int32)
    padded = jnp.where(rows[None, :] < NB,
                       both[:, jnp.minimum(rows, NB - 1)],
                       (rows % NV)[None, :])
    return padded.reshape(2, NW, N_CHUNKS, CHUNK)


def kernel(review_feats, user_feats, item_feats, user_idx, item_idx, W):
    w1 = W[:D]
    w23p = W[_PERM23]
    idx2 = _pad_idx2(user_idx, item_idx)
    up, ipj = _project_tables(user_feats, item_feats,
                              w23p[:D], w23p[D:])
    g = _sc_gather_add(up, ipj, idx2)
    return _final(review_feats, g, w1)


# speedup vs baseline: 2.3062x; 2.3062x over previous
import functools

import numpy as np
import jax
import jax.numpy as jnp
from jax import lax
from jax.experimental import pallas as pl
from jax.experimental.pallas import tpu as pltpu
from jax.experimental.pallas import tpu_sc as plsc

D = 128
NV = 50000
NB = 100000
NC, NS = 2, 16
NW = NC * NS
B_PAD = 102400
PER_W = B_PAD // NW
CHUNK = 128
N_CHUNKS = PER_W // CHUNK
NBUF = 3
LA = 2

_INV_PU = np.array([
    36, 58, 29, 7, 81, 105, 42, 113, 57, 115, 18, 3, 125, 93, 78, 102, 22,
    27, 10, 76, 60, 24, 95, 31, 87, 96, 127, 116, 92, 111, 101, 47, 40, 32,
    69, 28, 61, 122, 85, 37, 118, 51, 44, 34, 21, 79, 80, 73, 26, 119, 56,
    110, 52, 54, 1, 124, 67, 11, 41, 63, 12, 15, 23, 114, 121, 112, 45, 50,
    74, 108, 9, 75, 20, 48, 82, 86, 35, 38, 65, 59, 49, 55, 103, 97, 71, 33,
    5, 46, 4, 83, 106, 72, 70, 8, 0, 2, 98, 100, 84, 99, 25, 64, 94, 53,
    123, 13, 107, 43, 90, 6, 66, 89, 88, 17, 39, 77, 68, 104, 91, 126, 117,
    109, 14, 120, 19, 62, 16, 30], dtype=np.int32)
_INV_PI = np.array([
    102, 18, 20, 39, 35, 104, 13, 38, 87, 98, 82, 125, 103, 59, 33, 100,
    123, 26, 70, 42, 69, 99, 68, 90, 46, 56, 111, 63, 15, 121, 14, 126, 28,
    16, 110, 4, 113, 22, 65, 106, 57, 72, 54, 41, 62, 24, 48, 52, 29, 91,
    74, 107, 58, 21, 76, 124, 31, 12, 19, 0, 67, 79, 95, 17, 50, 45, 10, 96,
    120, 34, 23, 47, 73, 44, 92, 115, 32, 2, 75, 81, 117, 66, 97, 101, 105,
    53, 127, 83, 118, 108, 114, 71, 89, 36, 86, 1, 27, 11, 88, 77, 112, 85,
    5, 84, 49, 43, 60, 9, 37, 64, 8, 3, 109, 122, 55, 119, 61, 51, 30, 7,
    40, 80, 78, 116, 94, 25, 6, 93], dtype=np.int32)
_PERM23 = np.concatenate([_INV_PU + D, _INV_PI + 2 * D])


def _proj_body(u_ref, i_ref, w2_ref, w3_ref, up_ref, ip_ref):
    up_ref[...] = jnp.dot(u_ref[...], w2_ref[...],
                          preferred_element_type=jnp.float32)
    ip_ref[...] = jnp.dot(i_ref[...], w3_ref[...],
                          preferred_element_type=jnp.float32)


def _project_tables(user_feats, item_feats, w2p, w3p):
    blk = 10000
    return pl.pallas_call(
        _proj_body,
        grid=(NV // blk,),
        in_specs=[
            pl.BlockSpec((blk, D), lambda i: (i, 0)),
            pl.BlockSpec((blk, D), lambda i: (i, 0)),
            pl.BlockSpec((D, D), lambda i: (0, 0)),
            pl.BlockSpec((D, D), lambda i: (0, 0)),
        ],
        out_specs=[pl.BlockSpec((blk, D), lambda i: (i, 0)),
                   pl.BlockSpec((blk, D), lambda i: (i, 0))],
        out_shape=[jax.ShapeDtypeStruct((NV, D), jnp.float32),
                   jax.ShapeDtypeStruct((NV, D), jnp.float32)],
    )(user_feats, item_feats, w2p, w3p)


def _sc_gather_add(up, ipj, ui, ii):
    mesh = plsc.VectorSubcoreMesh(core_axis_name="c", subcore_axis_name="s")

    @functools.partial(
        pl.kernel,
        out_type=jax.ShapeDtypeStruct((B_PAD, D), jnp.float32),
        mesh=mesh,
        scratch_types=(
            [pltpu.VMEM((N_CHUNKS, CHUNK), jnp.int32),
             pltpu.VMEM((N_CHUNKS, CHUNK), jnp.int32),
             pltpu.VMEM((NBUF, CHUNK, D), jnp.float32),
             pltpu.VMEM((NBUF, CHUNK, D), jnp.float32)]
            + [pltpu.SemaphoreType.DMA] * (3 * NBUF)
        ),
    )
    def k(up_hbm, ip_hbm, ui_hbm, ii_hbm, g_hbm, idxu, idxi, rows_u, rows_i,
          *sems):
        sem_u = sems[0:NBUF]
        sem_i = sems[NBUF:2 * NBUF]
        sem_o = sems[2 * NBUF:3 * NBUF]
        wid = lax.axis_index("s") * NC + lax.axis_index("c")
        base = wid * PER_W
        pltpu.sync_copy(ui_hbm.at[wid], idxu)
        pltpu.sync_copy(ii_hbm.at[wid], idxi)

        g = [None] * N_CHUNKS
        w = [None] * N_CHUNKS

        def issue_g(j):
            b = j % NBUF
            g[j] = (pltpu.async_copy(up_hbm.at[idxu.at[j]], rows_u.at[b],
                                     sem_u[b]),
                    pltpu.async_copy(ip_hbm.at[idxi.at[j]], rows_i.at[b],
                                     sem_i[b]))

        for j in range(LA):
            issue_g(j)
        for j in range(N_CHUNKS):
            b = j % NBUF
            g[j][0].wait()
            g[j][1].wait()
            au = rows_u.at[b]
            ai = rows_i.at[b]

            def add_row(r, carry):
                for kk in range(D // 16):
                    s = pl.ds(kk * 16, 16)
                    au[r, s] = au[r, s] + ai[r, s]
                return carry

            lax.fori_loop(0, CHUNK, add_row, 0)
            w[j] = pltpu.async_copy(
                au, g_hbm.at[pl.ds(base + j * CHUNK, CHUNK)], sem_o[b])
            if j + LA < N_CHUNKS:
                if j + LA - NBUF >= 0:
                    w[j + LA - NBUF].wait()
                issue_g(j + LA)
        for j in range(N_CHUNKS + LA - NBUF, N_CHUNKS):
            w[j].wait()

    return k(up, ipj, ui, ii)


def _final_body(r_ref, g_ref, w1_ref, o_ref):
    acc = jnp.dot(r_ref[...], w1_ref[...], preferred_element_type=jnp.float32)
    o_ref[...] = jnp.maximum(acc + g_ref[...], 0.0)


def _final(review, g, w1):
    blk = 5000
    return pl.pallas_call(
        _final_body,
        grid=(NB // blk,),
        in_specs=[
            pl.BlockSpec((blk, D), lambda i: (i, 0)),
            pl.BlockSpec((blk, D), lambda i: (i, 0)),
            pl.BlockSpec((D, D), lambda i: (0, 0)),
        ],
        out_specs=pl.BlockSpec((blk, D), lambda i: (i, 0)),
        out_shape=jax.ShapeDtypeStruct((NB, D), jnp.float32),
    )(review, g, w1)


def _pad_idx(idx):
    pad = jnp.arange(B_PAD - NB, dtype=jnp.int32) % NV
    return jnp.concatenate([idx.astype(jnp.int32), pad]).reshape(
        NW, N_CHUNKS, CHUNK)


def kernel(review_feats, user_feats, item_feats, user_idx, item_idx, W):
    w1 = W[:D]
    w23p = W[_PERM23]
    w2p = w23p[:D]
    w3p = w23p[D:]
    ui = _pad_idx(user_idx)
    ii = _pad_idx(item_idx)
    up, ipj = _project_tables(user_feats, item_feats, w2p, w3p)
    g = _sc_gather_add(up, ipj, ui, ii)
    return _final(review_feats, g, w1)
